# baseline (speedup 1.0000x reference)
import functools

import jax
import jax.numpy as jnp
from jax import lax
from jax.experimental import pallas as pl
from jax.experimental.pallas import tpu as pltpu
from jax.experimental.pallas import tpu_sc as plsc

NC = 2
NS = 16
NW = NC * NS
LANES = 16
CH = 80
G_IDX = 6


def _deg_kernel(E, N):
    e_per_w = E // NW
    mesh = plsc.VectorSubcoreMesh(core_axis_name="c", subcore_axis_name="s")

    @functools.partial(
        pl.kernel,
        out_type=jax.ShapeDtypeStruct((NW, 1, N), jnp.float32),
        mesh=mesh,
        scratch_types=[
            pltpu.VMEM((e_per_w,), jnp.int32),
            pltpu.VMEM((N,), jnp.float32),
        ],
        compiler_params=pltpu.CompilerParams(needs_layout_passes=False),
    )
    def deg_kernel(dst_hbm, hist_hbm, dst_v, hist_v):
        c = lax.axis_index("c")
        s = lax.axis_index("s")
        wid = s * NC + c

        zero16 = jnp.zeros((LANES,), jnp.float32)

        def zbody(i, carry):
            hist_v[pl.ds(i * LANES, LANES)] = zero16
            return carry

        lax.fori_loop(0, N // LANES, zbody, 0)

        pltpu.sync_copy(dst_hbm.at[pl.ds(wid * e_per_w, e_per_w)], dst_v)

        ones16 = jnp.ones((LANES,), jnp.float32)

        def body(i, carry):
            idx = dst_v[pl.ds(i * LANES, LANES)]
            plsc.addupdate_scatter(hist_v, [idx], ones16)
            return carry

        lax.fori_loop(0, e_per_w // LANES, body, 0)

        pltpu.sync_copy(hist_v, hist_hbm.at[wid, 0])

    return deg_kernel


def _agg_kernel(N, D, E):
    e_per_w = E // NW
    n_chunks = e_per_w // CH
    rows_per_tile = ((N + NS - 1) // NS + 7) // 8 * 8
    n_pad = rows_per_tile * NS
    assert n_pad >= N + 1
    mesh = plsc.VectorSubcoreMesh(core_axis_name="c", subcore_axis_name="s")

    G = G_IDX
    gch = n_chunks // G
    assert n_chunks % G == 0 and gch % 3 == 0 and gch >= 9

    @functools.partial(
        pl.kernel,
        out_type=jax.ShapeDtypeStruct((NC, n_pad, D), jnp.float32),
        mesh=mesh,
        scratch_types=[
            pltpu.VMEM((2, gch, 1, CH), jnp.int32),
            pltpu.VMEM((2, gch, 1, CH), jnp.int32),
            pltpu.VMEM((CH, D), jnp.float32),
            pltpu.VMEM((CH, D), jnp.float32),
            pltpu.VMEM((CH, D), jnp.float32),
            pltpu.VMEM_SHARED((n_pad, D), jnp.float32),
            pltpu.SemaphoreType.DMA,
            pltpu.SemaphoreType.DMA,
            pltpu.SemaphoreType.DMA,
            pltpu.SemaphoreType.DMA,
        ],
        compiler_params=pltpu.CompilerParams(needs_layout_passes=False),
    )
    def agg_kernel(src_hbm, dst_hbm, y_hbm, out_hbm, sidx_v, didx_v, buf_0,
                   buf_1, buf_2, acc_sh, sem_0, sem_1, sem_2, sem_i):
        c = lax.axis_index("c")
        s = lax.axis_index("s")
        wid = s * NC + c
        row0 = s * rows_per_tile

        def idxfetch(g, b):
            sl = pl.ds(g * gch, gch)
            pltpu.async_copy(src_hbm.at[wid, sl], sidx_v.at[b], sem_i)
            pltpu.async_copy(dst_hbm.at[wid, sl], didx_v.at[b], sem_i)

        def idxwait(b):
            sl = pl.ds(0, gch)
            pltpu.make_async_copy(src_hbm.at[wid, sl], sidx_v.at[b], sem_i).wait()
            pltpu.make_async_copy(dst_hbm.at[wid, sl], didx_v.at[b], sem_i).wait()

        bufs = (buf_0, buf_1, buf_2)
        sems = (sem_0, sem_1, sem_2)

        def g_(b, i, r):
            pltpu.async_copy(y_hbm.at[sidx_v.at[b, i, 0]], bufs[r], sems[r])

        def s_(b, i, r):
            pltpu.async_copy(bufs[r], acc_sh.at[didx_v.at[b, i, 0]], sems[r],
                             add=True)

        def d_(r):
            pltpu.make_async_copy(y_hbm.at[pl.ds(0, CH)], bufs[r],
                                  sems[r]).wait()

        def step(b, i, r, nb=None, ni=None):
            d_(r)
            s_(b, i, r)
            if ni is not None:
                rn = (r + 1) % 3
                d_(rn)
                g_(nb, ni, rn)

        idxfetch(0, 0)

        zero16 = jnp.zeros((LANES,), jnp.float32)

        def zrow(r, carry):
            for k in range(D // LANES):
                buf_0[r, pl.ds(k * LANES, LANES)] = zero16
            return carry

        lax.fori_loop(0, CH, zrow, 0)

        full = rows_per_tile // CH
        rem = rows_per_tile % CH
        for j in range(full):
            pltpu.sync_copy(buf_0, acc_sh.at[pl.ds(row0 + j * CH, CH)])
        if rem:
            pltpu.sync_copy(
                buf_0.at[pl.ds(0, rem)],
                acc_sh.at[pl.ds(row0 + full * CH, rem)],
            )
        idxwait(0)
        plsc.subcore_barrier()

        g_(0, 0, 0)
        d_(0); s_(0, 0, 0); g_(0, 1, 1)
        d_(1); s_(0, 1, 1); g_(0, 2, 2)
        for g in range(G):
            b = g % 2
            if g + 1 < G:
                idxfetch(g + 1, 1 - b)
            base = 2 if g == 0 else 0
            n_tail = 4 if g == 0 else 3
            n_tri = (gch - base - n_tail) // 3

            def triple(j, carry, b=b, base=base):
                i0 = base + 3 * j
                for k in range(3):
                    step(b, i0 + k, (base + k) % 3, b, i0 + k + 1)
                return carry

            lax.fori_loop(0, n_tri, triple, 0)
            for i in range(gch - n_tail, gch - 1):
                step(b, i, i % 3, b, i + 1)
            i_last = gch - 1
            r_last = i_last % 3
            if g + 1 < G:
                idxwait(1 - b)
                step(b, i_last, r_last, 1 - b, 0)
            else:
                step(b, i_last, r_last)
                d_((r_last + 1) % 3)
        d_((gch - 2) % 3)
        d_((gch - 1) % 3)
        plsc.subcore_barrier()

        pltpu.sync_copy(
            acc_sh.at[pl.ds(row0, rows_per_tile)],
            out_hbm.at[c, pl.ds(row0, rows_per_tile)],
        )

    return agg_kernel


def _tc_transform(x, W, histT):
    N, _ = x.shape
    Dout = W.shape[1]

    def body(x_ref, w_ref, h_ref, y_ref, dis_ref):
        deg = jnp.sum(h_ref[...], axis=1, keepdims=True) + 1.0
        dis = lax.rsqrt(deg)
        xw = jnp.dot(x_ref[...], w_ref[...], preferred_element_type=jnp.float32)
        y_ref[...] = xw * dis
        dis_ref[...] = dis

    return pl.pallas_call(
        body,
        out_shape=(
            jax.ShapeDtypeStruct((N, Dout), jnp.float32),
            jax.ShapeDtypeStruct((N, 1), jnp.float32),
        ),
    )(x, W, histT)


def _tc_combine(agg, y, dis):
    N, D = y.shape

    def body(a_ref, y_ref, d_ref, o_ref):
        a = (a_ref[0] + a_ref[1])[:N]
        o_ref[...] = d_ref[...] * (a + y_ref[...])

    return pl.pallas_call(
        body,
        out_shape=jax.ShapeDtypeStruct((N, D), jnp.float32),
    )(agg, y, dis)


def kernel(x, edge_index, W):
    N, _ = x.shape
    Dout = W.shape[1]
    E = edge_index.shape[1]
    assert E % NW == 0 and (E // NW) % LANES == 0
    assert N % NS == 0 and N % LANES == 0 and Dout % LANES == 0

    ei = edge_index.astype(jnp.int32)
    srcs = ei[0]
    dsts = ei[1]

    hist = _deg_kernel(E, N)(dsts)
    histT = hist.reshape(NW, N).T
    y, dis = _tc_transform(x, W, histT)

    e_per_w = E // NW
    n_chunks = -(-e_per_w // CH)
    n_chunks = -(-n_chunks // G_IDX) * G_IDX
    ppw = n_chunks * CH - e_per_w
    rows_per_tile = ((N + NS - 1) // NS + 7) // 8 * 8
    n_spare = rows_per_tile * NS - N
    pad_src = jnp.zeros((NW, ppw), jnp.int32)
    pad_dst = N + (
        jnp.arange(NW, dtype=jnp.int32)[:, None]
        + NW * (jnp.arange(max(ppw, 1), dtype=jnp.int32)[None, :ppw] % 3)
    ) % n_spare
    src3 = jnp.concatenate([srcs.reshape(NW, e_per_w), pad_src], axis=1)
    dst3 = jnp.concatenate([dsts.reshape(NW, e_per_w), pad_dst], axis=1)
    src3 = src3.reshape(NW, n_chunks, 1, CH)
    dst3 = dst3.reshape(NW, n_chunks, 1, CH)
    agg = _agg_kernel(N, Dout, NW * n_chunks * CH)(src3, dst3, y)
    return _tc_combine(agg, y, dis)


# speedup vs baseline: 1.6814x; 1.6814x over previous
import functools

import jax
import jax.numpy as jnp
from jax import lax
from jax.experimental import pallas as pl
from jax.experimental.pallas import tpu as pltpu
from jax.experimental.pallas import tpu_sc as plsc

NC = 2
NS = 16
NW = NC * NS
LANES = 16
CH = 80
G_IDX = 5


def _deg_kernel(E, N):
    e_per_w = E // NW
    mesh = plsc.VectorSubcoreMesh(core_axis_name="c", subcore_axis_name="s")

    @functools.partial(
        pl.kernel,
        out_type=jax.ShapeDtypeStruct((NW, 1, N), jnp.float32),
        mesh=mesh,
        scratch_types=[
            pltpu.VMEM((e_per_w,), jnp.int32),
            pltpu.VMEM((N,), jnp.float32),
        ],
        compiler_params=pltpu.CompilerParams(needs_layout_passes=False),
    )
    def deg_kernel(dst_hbm, hist_hbm, dst_v, hist_v):
        c = lax.axis_index("c")
        s = lax.axis_index("s")
        wid = s * NC + c

        zero16 = jnp.zeros((LANES,), jnp.float32)

        def zbody(i, carry):
            hist_v[pl.ds(i * LANES, LANES)] = zero16
            return carry

        lax.fori_loop(0, N // LANES, zbody, 0)

        pltpu.sync_copy(dst_hbm.at[pl.ds(wid * e_per_w, e_per_w)], dst_v)

        ones16 = jnp.ones((LANES,), jnp.float32)

        def body(i, carry):
            idx = dst_v[pl.ds(i * LANES, LANES)]
            plsc.addupdate_scatter(hist_v, [idx], ones16)
            return carry

        lax.fori_loop(0, e_per_w // LANES, body, 0)

        pltpu.sync_copy(hist_v, hist_hbm.at[wid, 0])

    return deg_kernel


def _agg_kernel(N, D, E):
    e_per_w = E // NW
    n_chunks = e_per_w // CH
    rows_per_tile = ((N + NS - 1) // NS + 7) // 8 * 8
    n_pad = rows_per_tile * NS
    assert n_pad >= N + 1
    mesh = plsc.VectorSubcoreMesh(core_axis_name="c", subcore_axis_name="s")

    G = G_IDX
    gch = n_chunks // G
    assert n_chunks % G == 0 and gch % 2 == 1

    @functools.partial(
        pl.kernel,
        out_type=jax.ShapeDtypeStruct((NC, n_pad, D), jnp.float32),
        mesh=mesh,
        scratch_types=[
            pltpu.VMEM((2, gch, 1, CH), jnp.int32),
            pltpu.VMEM((2, gch, 1, CH), jnp.int32),
            pltpu.VMEM((CH, D), jnp.float32),
            pltpu.VMEM((CH, D), jnp.float32),
            pltpu.VMEM_SHARED((n_pad, D), jnp.float32),
            pltpu.SemaphoreType.DMA,
            pltpu.SemaphoreType.DMA,
            pltpu.SemaphoreType.DMA,
        ],
        compiler_params=pltpu.CompilerParams(needs_layout_passes=False),
    )
    def agg_kernel(src_hbm, dst_hbm, y_hbm, out_hbm, sidx_v, didx_v, buf_a,
                   buf_b, acc_sh, sem_a, sem_b, sem_i):
        c = lax.axis_index("c")
        s = lax.axis_index("s")
        wid = s * NC + c
        row0 = s * rows_per_tile

        def idxfetch(g, b):
            sl = pl.ds(g * gch, gch)
            pltpu.async_copy(src_hbm.at[wid, sl], sidx_v.at[b], sem_i)
            pltpu.async_copy(dst_hbm.at[wid, sl], didx_v.at[b], sem_i)

        def idxwait(b):
            sl = pl.ds(0, gch)
            pltpu.make_async_copy(src_hbm.at[wid, sl], sidx_v.at[b], sem_i).wait()
            pltpu.make_async_copy(dst_hbm.at[wid, sl], didx_v.at[b], sem_i).wait()

        def gather(b, i, buf, sem):
            pltpu.async_copy(y_hbm.at[sidx_v.at[b, i, 0]], buf, sem)

        def drain(buf, sem):
            pltpu.make_async_copy(y_hbm.at[pl.ds(0, CH)], buf, sem).wait()

        def scatter(b, i, buf):
            pltpu.sync_copy(buf, acc_sh.at[didx_v.at[b, i, 0]], add=True)

        idxfetch(0, 0)

        zero16 = jnp.zeros((LANES,), jnp.float32)

        def zrow(r, carry):
            for k in range(D // LANES):
                buf_a[r, pl.ds(k * LANES, LANES)] = zero16
            return carry

        lax.fori_loop(0, CH, zrow, 0)

        full = rows_per_tile // CH
        rem = rows_per_tile % CH
        for j in range(full):
            pltpu.sync_copy(buf_a, acc_sh.at[pl.ds(row0 + j * CH, CH)])
        if rem:
            pltpu.sync_copy(
                buf_a.at[pl.ds(0, rem)],
                acc_sh.at[pl.ds(row0 + full * CH, rem)],
            )
        idxwait(0)
        plsc.subcore_barrier()

        cur, nxt = buf_a, buf_b
        scur, snxt = sem_a, sem_b
        gather(0, 0, cur, scur)
        for g in range(G):
            b = g % 2
            if g + 1 < G:
                idxfetch(g + 1, 1 - b)

            def pair(j, carry, b=b, cur=cur, nxt=nxt, scur=scur, snxt=snxt):
                i0 = 2 * j
                gather(b, i0 + 1, nxt, snxt)
                drain(cur, scur)
                scatter(b, i0, cur)
                gather(b, i0 + 2, cur, scur)
                drain(nxt, snxt)
                scatter(b, i0 + 1, nxt)
                return carry

            lax.fori_loop(0, (gch - 1) // 2, pair, 0)
            if g + 1 < G:
                idxwait(1 - b)
                gather(1 - b, 0, nxt, snxt)
            drain(cur, scur)
            scatter(b, gch - 1, cur)
            cur, nxt = nxt, cur
            scur, snxt = snxt, scur
        plsc.subcore_barrier()

        pltpu.sync_copy(
            acc_sh.at[pl.ds(row0, rows_per_tile)],
            out_hbm.at[c, pl.ds(row0, rows_per_tile)],
        )

    return agg_kernel


def _tc_transform(x, W, histT):
    N, _ = x.shape
    Dout = W.shape[1]

    def body(x_ref, w_ref, h_ref, y_ref, dis_ref):
        deg = jnp.sum(h_ref[...], axis=1, keepdims=True) + 1.0
        dis = lax.rsqrt(deg)
        xw = jnp.dot(x_ref[...], w_ref[...], preferred_element_type=jnp.float32)
        y_ref[...] = xw * dis
        dis_ref[...] = dis

    return pl.pallas_call(
        body,
        out_shape=(
            jax.ShapeDtypeStruct((N, Dout), jnp.float32),
            jax.ShapeDtypeStruct((N, 1), jnp.float32),
        ),
    )(x, W, histT)


def _tc_combine(agg, y, dis):
    N, D = y.shape

    def body(a_ref, y_ref, d_ref, o_ref):
        a = (a_ref[0] + a_ref[1])[:N]
        o_ref[...] = d_ref[...] * (a + y_ref[...])

    return pl.pallas_call(
        body,
        out_shape=jax.ShapeDtypeStruct((N, D), jnp.float32),
    )(agg, y, dis)


def kernel(x, edge_index, W):
    N, _ = x.shape
    Dout = W.shape[1]
    E = edge_index.shape[1]
    assert E % NW == 0 and (E // NW) % LANES == 0
    assert N % NS == 0 and N % LANES == 0 and Dout % LANES == 0

    ei = edge_index.astype(jnp.int32)
    srcs = ei[0]
    dsts = ei[1]

    hist = _deg_kernel(E, N)(dsts)
    histT = hist.reshape(NW, N).T
    y, dis = _tc_transform(x, W, histT)

    e_per_w = E // NW
    n_chunks = -(-e_per_w // CH)
    n_chunks = -(-n_chunks // G_IDX) * G_IDX
    ppw = n_chunks * CH - e_per_w
    rows_per_tile = ((N + NS - 1) // NS + 7) // 8 * 8
    n_spare = rows_per_tile * NS - N
    pad_src = jnp.zeros((NW, ppw), jnp.int32)
    pad_dst = jnp.broadcast_to(
        N + (jnp.arange(NW, dtype=jnp.int32) % n_spare)[:, None], (NW, ppw)
    )
    src3 = jnp.concatenate([srcs.reshape(NW, e_per_w), pad_src], axis=1)
    dst3 = jnp.concatenate([dsts.reshape(NW, e_per_w), pad_dst], axis=1)
    src3 = src3.reshape(NW, n_chunks, 1, CH)
    dst3 = dst3.reshape(NW, n_chunks, 1, CH)
    agg = _agg_kernel(N, Dout, NW * n_chunks * CH)(src3, dst3, y)
    return _tc_combine(agg, y, dis)
